# Initial kernel scaffold; baseline (speedup 1.0000x reference)
import jax
import jax.numpy as jnp
from jax.experimental import pallas as pl


def kernel(input):
    raise NotImplementedError("write your pallas kernel here")


# speedup vs baseline: 1.2386x; 1.2386x over previous
import jax
import jax.numpy as jnp
from jax.experimental import pallas as pl
from jax.experimental.pallas import tpu as pltpu

N = 32
F = 7
KP1 = N - F - 2 + 1
D = 1048576
BLK = 16384
NB = D // BLK


def _krum_body(x_ref, o_ref, acc_ref, w_ref):
    i = pl.program_id(0)
    nb = pl.num_programs(0) // 2

    @pl.when(i == 0)
    def _init():
        acc_ref[...] = jnp.zeros_like(acc_ref)

    @pl.when(i < nb)
    def _phase1():
        blk = x_ref[...]
        acc_ref[...] += jax.lax.dot_general(
            blk, blk, (((0,), (0,)), ((), ())),
            preferred_element_type=jnp.float32)

    @pl.when(i == nb)
    def _select():
        gram = acc_ref[...]
        row_ids = jax.lax.broadcasted_iota(jnp.int32, (N, N), 0)
        col_ids = jax.lax.broadcasted_iota(jnp.int32, (N, N), 1)
        eye = row_ids == col_ids
        diag_r = jnp.sum(jnp.where(eye, gram, 0.0), axis=1, keepdims=True)
        diag_c = jnp.sum(jnp.where(eye, gram, 0.0), axis=0, keepdims=True)
        d2 = diag_r + diag_c - 2.0 * gram
        cdist = jnp.sqrt(jnp.maximum(d2, 1e-12))

        work = cdist
        sel = jnp.zeros((N, N), jnp.float32)
        for _ in range(KP1):
            m = jnp.min(work, axis=1, keepdims=True)
            cand = jnp.where(work == m, col_ids, N)
            jmin = jnp.min(cand, axis=1, keepdims=True)
            chosen = col_ids == jmin
            sel = sel + chosen.astype(jnp.float32)
            work = jnp.where(chosen, jnp.inf, work)

        row_sum = jnp.sum(cdist * sel, axis=1, keepdims=True)
        ms = jnp.min(row_sum)
        cand_i = jnp.where(row_sum == ms, row_ids[:, :1], N)
        i_star = jnp.min(cand_i)
        sel_row = jnp.where(row_ids == i_star, sel, 0.0)
        w_ref[...] = jnp.sum(sel_row, axis=0, keepdims=True) * (1.0 / KP1)

    @pl.when(i >= nb)
    def _phase2():
        blk = x_ref[...]
        w = w_ref[...]
        res = jnp.sum(blk * w, axis=1)
        o_ref[...] = jnp.reshape(res, (BLK // 128, 128))


def kernel(input):
    x2 = input.reshape(D, N)
    out = pl.pallas_call(
        _krum_body,
        grid=(2 * NB,),
        in_specs=[pl.BlockSpec((BLK, N), lambda i: (jax.lax.rem(i, NB), 0))],
        out_specs=pl.BlockSpec(
            (BLK // 128, 128),
            lambda i: (jnp.where(i < NB, 0, i - NB), 0)),
        out_shape=jax.ShapeDtypeStruct((D // 128, 128), jnp.float32),
        scratch_shapes=[
            pltpu.VMEM((N, N), jnp.float32),
            pltpu.VMEM((1, N), jnp.float32),
        ],
    )(x2)
    return out.reshape(1, D, 1)


# speedup vs baseline: 6.7938x; 5.4849x over previous
import jax
import jax.numpy as jnp
from jax.experimental import pallas as pl
from jax.experimental.pallas import tpu as pltpu

N = 32
F = 7
KP1 = N - F - 2 + 1
D = 1048576
BLKD = 65536
NB = D // BLKD


def _krum_body(x_ref, o_ref, acc_ref, sq_ref, w_ref):
    i = pl.program_id(0)
    nb = pl.num_programs(0) // 2

    @pl.when(i == 0)
    def _init():
        acc_ref[...] = jnp.zeros_like(acc_ref)
        sq_ref[...] = jnp.zeros_like(sq_ref)

    @pl.when(i < nb)
    def _phase1():
        blk = x_ref[...]
        blk16 = blk.astype(jnp.bfloat16)
        acc_ref[...] += jax.lax.dot_general(
            blk16, blk16, (((1,), (1,)), ((), ())),
            preferred_element_type=jnp.float32)
        sq_ref[...] += jnp.sum(blk * blk, axis=1, keepdims=True)

    @pl.when(i == nb)
    def _select():
        gram = acc_ref[...]
        row_ids = jax.lax.broadcasted_iota(jnp.int32, (N, N), 0)
        col_ids = jax.lax.broadcasted_iota(jnp.int32, (N, N), 1)
        eye = row_ids == col_ids
        sq_col = sq_ref[...]
        sq_row = jnp.sum(jnp.where(eye, sq_col, 0.0), axis=0,
                         keepdims=True)
        d2 = sq_col + sq_row - 2.0 * gram
        cdist = jnp.sqrt(jnp.maximum(d2, 1e-12))

        work = cdist
        sel = jnp.zeros((N, N), jnp.float32)
        for _ in range(KP1):
            m = jnp.min(work, axis=0, keepdims=True)
            cand = jnp.where(work == m, row_ids, N)
            jmin = jnp.min(cand, axis=0, keepdims=True)
            chosen = row_ids == jmin
            sel = sel + chosen.astype(jnp.float32)
            work = jnp.where(chosen, jnp.inf, work)

        col_sum = jnp.sum(cdist * sel, axis=0, keepdims=True)
        ms = jnp.min(col_sum)
        cand_i = jnp.where(col_sum == ms, col_ids[:1, :], N)
        i_star = jnp.min(cand_i)
        sel_col = jnp.where(col_ids == i_star, sel, 0.0)
        w_ref[...] = jnp.sum(sel_col, axis=1, keepdims=True) * (1.0 / KP1)

    @pl.when(i >= nb)
    def _phase2():
        blk = x_ref[...]
        w = w_ref[...]
        res = jnp.sum(blk * w, axis=0, keepdims=True)
        o_ref[...] = res.reshape(1, 1, BLKD)


def kernel(input):
    xt = jnp.transpose(input, (0, 2, 1)).reshape(N, D)
    out = pl.pallas_call(
        _krum_body,
        grid=(2 * NB,),
        in_specs=[pl.BlockSpec((N, BLKD), lambda i: (0, jax.lax.rem(i, NB)))],
        out_specs=pl.BlockSpec(
            (1, 1, BLKD),
            lambda i: (jnp.where(i < NB, 0, i - NB), 0, 0)),
        out_shape=jax.ShapeDtypeStruct((NB, 1, BLKD), jnp.float32),
        scratch_shapes=[
            pltpu.VMEM((N, N), jnp.float32),
            pltpu.VMEM((N, 1), jnp.float32),
            pltpu.VMEM((N, 1), jnp.float32),
        ],
    )(xt)
    return out.reshape(1, D, 1)
